# baseline (speedup 1.0000x reference)
import functools

import jax
import jax.numpy as jnp
from jax import lax
from jax.experimental import pallas as pl
from jax.experimental.pallas import tpu as pltpu
from jax.experimental.pallas import tpu_sc as plsc

B = 16384
D = 32
NC = 2
NS = 16
NW = NC * NS
RPW = B // NW
CHUNK = 128
NCHUNK = RPW // CHUNK
LANES = 16


def _gmf_body(vidx_hbm, hidx_hbm, virus_hbm, human_hbm, par_hbm, out_hbm,
              vidx_v, hidx_v, u_rows, v_rows, par_v, res_v, sem):
    wid = lax.axis_index("s") * NC + lax.axis_index("c")
    base = wid * RPW

    pltpu.sync_copy(vidx_hbm.at[pl.ds(wid * NCHUNK, NCHUNK)], vidx_v)
    pltpu.sync_copy(hidx_hbm.at[pl.ds(wid * NCHUNK, NCHUNK)], hidx_v)
    pltpu.sync_copy(par_hbm, par_v)

    handles = []
    for j in range(NCHUNK):
        handles.append(pltpu.async_copy(
            virus_hbm.at[vidx_v.at[j]], u_rows.at[pl.ds(j * CHUNK, CHUNK)], sem))
        handles.append(pltpu.async_copy(
            human_hbm.at[hidx_v.at[j]], v_rows.at[pl.ds(j * CHUNK, CHUNK)], sem))
    for h in handles:
        h.wait()

    wa = par_v[pl.ds(0, LANES)]
    wb = par_v[pl.ds(LANES, LANES)]
    w_scal = [wa[d] for d in range(LANES)] + [wb[d] for d in range(LANES)]
    bvec = par_v[pl.ds(D, LANES)]

    def group(g, carry):
        rows = g * LANES + jnp.arange(LANES, dtype=jnp.int32)
        acc = bvec
        for d in range(D):
            dcol = jnp.full((LANES,), d, dtype=jnp.int32)
            ud = plsc.load_gather(u_rows, [rows, dcol])
            vd = plsc.load_gather(v_rows, [rows, dcol])
            acc = acc + ud * vd * w_scal[d]
        res_v[pl.ds(g * LANES, LANES)] = 1.0 / (1.0 + jnp.exp(-acc))
        return carry

    lax.fori_loop(0, RPW // LANES, group, 0)

    pltpu.sync_copy(res_v, out_hbm.at[pl.ds(base, RPW)])


@jax.jit
def _gmf(vidx2d, hidx2d, virus_table, human_table, params):
    mesh = plsc.VectorSubcoreMesh(core_axis_name="c", subcore_axis_name="s")
    return pl.kernel(
        _gmf_body,
        out_type=jax.ShapeDtypeStruct((B,), jnp.float32),
        mesh=mesh,
        compiler_params=pltpu.CompilerParams(
            needs_layout_passes=False, use_tc_tiling_on_sc=False),
        scratch_types=[
            pltpu.VMEM((NCHUNK, CHUNK), jnp.int32),
            pltpu.VMEM((NCHUNK, CHUNK), jnp.int32),
            pltpu.VMEM((RPW, D), jnp.float32),
            pltpu.VMEM((RPW, D), jnp.float32),
            pltpu.VMEM((D + LANES,), jnp.float32),
            pltpu.VMEM((RPW,), jnp.float32),
            pltpu.SemaphoreType.DMA,
        ],
    )(vidx2d, hidx2d, virus_table, human_table, params)


def kernel(v_idxs, h_idxs, virus_table, human_table, W, b):
    vidx2d = v_idxs.astype(jnp.int32).reshape(NW * NCHUNK, CHUNK)
    hidx2d = h_idxs.astype(jnp.int32).reshape(NW * NCHUNK, CHUNK)
    params = jnp.concatenate(
        [W.reshape(D).astype(jnp.float32),
         jnp.broadcast_to(b.astype(jnp.float32), (LANES,))])
    out = _gmf(vidx2d, hidx2d, virus_table, human_table, params)
    return out.reshape(B, 1)


# speedup vs baseline: 1.5157x; 1.5157x over previous
import functools

import jax
import jax.numpy as jnp
from jax import lax
from jax.experimental import pallas as pl
from jax.experimental.pallas import tpu as pltpu
from jax.experimental.pallas import tpu_sc as plsc

B = 16384
D = 32
NC = 2
NS = 16
NW = NC * NS
RPW = B // NW
CHUNK = 32
NITER = RPW // CHUNK
LANES = 16


def _gmf_body(vidx_hbm, hidx_hbm, virus_hbm, human_hbm, par_hbm, out_hbm,
              vidx_v, hidx_v, u_rows, v_rows, par_v, res_v, sem):
    wid = lax.axis_index("s") * NC + lax.axis_index("c")
    base = wid * RPW

    pltpu.sync_copy(vidx_hbm.at[pl.ds(base, RPW)], vidx_v)
    pltpu.sync_copy(hidx_hbm.at[pl.ds(base, RPW)], hidx_v)
    pltpu.sync_copy(par_hbm, par_v)

    wa = par_v[pl.ds(0, LANES)]
    wb = par_v[pl.ds(LANES, LANES)]
    w_scal = [wa[d] for d in range(LANES)] + [wb[d] for d in range(LANES)]
    bvec = par_v[pl.ds(D, LANES)]
    slot_iota = jnp.arange(LANES, dtype=jnp.int32)

    def step(i, carry):
        handles = []
        for g in range(CHUNK // LANES):
            iv = vidx_v[pl.ds(i * CHUNK + g * LANES, LANES)]
            ih = hidx_v[pl.ds(i * CHUNK + g * LANES, LANES)]
            for l in range(LANES):
                slot = g * LANES + l
                handles.append(pltpu.async_copy(
                    virus_hbm.at[pl.ds(iv[l], 1)],
                    u_rows.at[pl.ds(slot, 1)], sem))
                handles.append(pltpu.async_copy(
                    human_hbm.at[pl.ds(ih[l], 1)],
                    v_rows.at[pl.ds(slot, 1)], sem))
        for h in handles:
            h.wait()
        for g in range(CHUNK // LANES):
            slot = g * LANES + slot_iota
            acc = bvec
            for d in range(D):
                dcol = jnp.full((LANES,), d, dtype=jnp.int32)
                ud = plsc.load_gather(u_rows, [slot, dcol])
                vd = plsc.load_gather(v_rows, [slot, dcol])
                acc = acc + ud * vd * w_scal[d]
            res_v[pl.ds(i * CHUNK + g * LANES, LANES)] = (
                1.0 / (1.0 + jnp.exp(-acc)))
        return carry

    lax.fori_loop(0, NITER, step, 0)

    pltpu.sync_copy(res_v, out_hbm.at[pl.ds(base, RPW)])


@jax.jit
def _gmf(v_idxs, h_idxs, virus_table, human_table, params):
    mesh = plsc.VectorSubcoreMesh(core_axis_name="c", subcore_axis_name="s")
    return pl.kernel(
        _gmf_body,
        out_type=jax.ShapeDtypeStruct((B,), jnp.float32),
        mesh=mesh,
        compiler_params=pltpu.CompilerParams(
            needs_layout_passes=False, use_tc_tiling_on_sc=True),
        scratch_types=[
            pltpu.VMEM((RPW,), jnp.int32),
            pltpu.VMEM((RPW,), jnp.int32),
            pltpu.VMEM((CHUNK, D), jnp.float32),
            pltpu.VMEM((CHUNK, D), jnp.float32),
            pltpu.VMEM((D + LANES,), jnp.float32),
            pltpu.VMEM((RPW,), jnp.float32),
            pltpu.SemaphoreType.DMA,
        ],
    )(v_idxs, h_idxs, virus_table, human_table, params)


def kernel(v_idxs, h_idxs, virus_table, human_table, W, b):
    params = jnp.concatenate(
        [W.reshape(D).astype(jnp.float32),
         jnp.broadcast_to(b.astype(jnp.float32), (LANES,))])
    out = _gmf(v_idxs.astype(jnp.int32), h_idxs.astype(jnp.int32),
               virus_table, human_table, params)
    return out.reshape(B, 1)


# speedup vs baseline: 1.5163x; 1.0004x over previous
import functools

import jax
import jax.numpy as jnp
from jax import lax
from jax.experimental import pallas as pl
from jax.experimental.pallas import tpu as pltpu
from jax.experimental.pallas import tpu_sc as plsc

B = 16384
D = 32
NC = 2
NS = 16
NW = NC * NS
RPW = B // NW
CHUNK = 32
NITER = RPW // CHUNK
LANES = 16


def _gmf_body(vidx_hbm, hidx_hbm, virus_hbm, human_hbm, par_hbm, out_hbm,
              vidx_v, hidx_v, u_rows, v_rows, par_v, res_v,
              sem0, sem1, sem2, sem3):
    sems = [sem0, sem1, sem2, sem3]
    wid = lax.axis_index("s") * NC + lax.axis_index("c")
    base = wid * RPW

    pltpu.sync_copy(vidx_hbm.at[pl.ds(base, RPW)], vidx_v)
    pltpu.sync_copy(hidx_hbm.at[pl.ds(base, RPW)], hidx_v)
    pltpu.sync_copy(par_hbm, par_v)

    wa = par_v[pl.ds(0, LANES)]
    wb = par_v[pl.ds(LANES, LANES)]
    w_scal = [wa[d] for d in range(LANES)] + [wb[d] for d in range(LANES)]
    bvec = par_v[pl.ds(D, LANES)]
    slot_iota = jnp.arange(LANES, dtype=jnp.int32)

    def step(i, carry):
        handles = []
        for g in range(CHUNK // LANES):
            iv = vidx_v[pl.ds(i * CHUNK + g * LANES, LANES)]
            ih = hidx_v[pl.ds(i * CHUNK + g * LANES, LANES)]
            for l in range(LANES):
                slot = g * LANES + l
                handles.append(pltpu.async_copy(
                    virus_hbm.at[pl.ds(iv[l], 1)],
                    u_rows.at[pl.ds(slot, 1)], sems[slot % 4]))
                handles.append(pltpu.async_copy(
                    human_hbm.at[pl.ds(ih[l], 1)],
                    v_rows.at[pl.ds(slot, 1)], sems[(slot + 2) % 4]))
        for h in handles:
            h.wait()
        for g in range(CHUNK // LANES):
            slot = g * LANES + slot_iota
            acc = bvec
            for d in range(D):
                dcol = jnp.full((LANES,), d, dtype=jnp.int32)
                ud = plsc.load_gather(u_rows, [slot, dcol])
                vd = plsc.load_gather(v_rows, [slot, dcol])
                acc = acc + ud * vd * w_scal[d]
            res_v[pl.ds(i * CHUNK + g * LANES, LANES)] = (
                1.0 / (1.0 + jnp.exp(-acc)))
        return carry

    lax.fori_loop(0, NITER, step, 0)

    pltpu.sync_copy(res_v, out_hbm.at[pl.ds(base, RPW)])


@jax.jit
def _gmf(v_idxs, h_idxs, virus_table, human_table, params):
    mesh = plsc.VectorSubcoreMesh(core_axis_name="c", subcore_axis_name="s")
    return pl.kernel(
        _gmf_body,
        out_type=jax.ShapeDtypeStruct((B,), jnp.float32),
        mesh=mesh,
        compiler_params=pltpu.CompilerParams(
            needs_layout_passes=False, use_tc_tiling_on_sc=True),
        scratch_types=[
            pltpu.VMEM((RPW,), jnp.int32),
            pltpu.VMEM((RPW,), jnp.int32),
            pltpu.VMEM((CHUNK, D), jnp.float32),
            pltpu.VMEM((CHUNK, D), jnp.float32),
            pltpu.VMEM((D + LANES,), jnp.float32),
            pltpu.VMEM((RPW,), jnp.float32),
            pltpu.SemaphoreType.DMA,
            pltpu.SemaphoreType.DMA,
            pltpu.SemaphoreType.DMA,
            pltpu.SemaphoreType.DMA,
        ],
    )(v_idxs, h_idxs, virus_table, human_table, params)


def kernel(v_idxs, h_idxs, virus_table, human_table, W, b):
    params = jnp.concatenate(
        [W.reshape(D).astype(jnp.float32),
         jnp.broadcast_to(b.astype(jnp.float32), (LANES,))])
    out = _gmf(v_idxs.astype(jnp.int32), h_idxs.astype(jnp.int32),
               virus_table, human_table, params)
    return out.reshape(B, 1)
